# baseline (speedup 1.0000x reference)
import functools
import jax
import jax.numpy as jnp
from jax.experimental import pallas as pl
from jax.experimental.pallas import tpu as pltpu

D = 1024
E = 8
K = 2
I = 2048
NL = 4
T = 4096

TB_R = 1024
TB = 512
TI = 512
IT = I // TI


def _router_body(x_ref, gw_ref, ids_ref, w_ref, aux_ref, usage_acc, prob_acc):
    step = pl.program_id(0)
    nsteps = pl.num_programs(0)
    logits = jax.lax.dot_general(x_ref[...], gw_ref[...],
                                 (((1,), (1,)), ((), ())),
                                 preferred_element_type=jnp.float32)
    m = jnp.max(logits, axis=1, keepdims=True)
    p = jnp.exp(logits - m)
    s = jnp.sum(p, axis=1, keepdims=True)
    probs = p / s

    top1v = jnp.max(probs, axis=1)
    top1i = jnp.argmax(probs, axis=1).astype(jnp.int32)
    cols = jax.lax.broadcasted_iota(jnp.int32, probs.shape, 1)
    masked = jnp.where(cols == top1i[:, None], -jnp.inf, probs)
    top2v = jnp.max(masked, axis=1)
    top2i = jnp.argmax(masked, axis=1).astype(jnp.int32)

    denom = top1v + top2v + 1e-9
    ids_ref[...] = jnp.stack([top1i, top2i], axis=1)
    w_ref[...] = jnp.stack([top1v / denom, top2v / denom], axis=1)

    usage = jnp.sum((cols == top1i[:, None]).astype(jnp.float32), axis=0,
                    keepdims=True)
    psum = jnp.sum(probs, axis=0, keepdims=True)

    @pl.when(step == 0)
    def _():
        usage_acc[...] = jnp.zeros_like(usage_acc)
        prob_acc[...] = jnp.zeros_like(prob_acc)

    usage_acc[...] += usage
    prob_acc[...] += psum

    @pl.when(step == nsteps - 1)
    def _():
        aux_ref[...] = jnp.reshape(
            E * jnp.sum((usage_acc[...] / T) * (prob_acc[...] / T)), (1, 1))


def _router(x, gate_w):
    grid = (T // TB_R,)
    return pl.pallas_call(
        _router_body,
        grid=grid,
        in_specs=[
            pl.BlockSpec((TB_R, D), lambda i: (i, 0)),
            pl.BlockSpec((E, D), lambda i: (0, 0)),
        ],
        out_specs=[
            pl.BlockSpec((TB_R, K), lambda i: (i, 0)),
            pl.BlockSpec((TB_R, K), lambda i: (i, 0)),
            pl.BlockSpec((1, 1), lambda i: (0, 0)),
        ],
        out_shape=[
            jax.ShapeDtypeStruct((T, K), jnp.int32),
            jax.ShapeDtypeStruct((T, K), jnp.float32),
            jax.ShapeDtypeStruct((1, 1), jnp.float32),
        ],
        scratch_shapes=[
            pltpu.VMEM((1, E), jnp.float32),
            pltpu.VMEM((1, E), jnp.float32),
        ],
    )(x, gate_w)


def _expert_body(ids_ref, w_ref, x_ref, wg_ref, wu_ref, wd_ref, out_ref,
                 acc, tokacc):
    e = pl.program_id(1)
    it = pl.program_id(2)

    @pl.when(it == 0)
    def _():
        acc[...] = jnp.zeros_like(acc)

    x = x_ref[...]
    g = jax.lax.dot_general(x, wg_ref[0], (((1,), (1,)), ((), ())),
                            preferred_element_type=jnp.float32)
    u = jax.lax.dot_general(x, wu_ref[0], (((1,), (1,)), ((), ())),
                            preferred_element_type=jnp.float32)
    h = g * jax.nn.sigmoid(g) * u
    acc[...] += jax.lax.dot_general(h, wd_ref[0], (((1,), (1,)), ((), ())),
                                    preferred_element_type=jnp.float32)

    @pl.when(it == IT - 1)
    def _():
        we = jnp.sum(w_ref[...] * (ids_ref[...] == e).astype(jnp.float32),
                     axis=1)
        contrib = acc[...] * we[:, None]

        @pl.when(e == 0)
        def _():
            tokacc[...] = contrib

        @pl.when(e > 0)
        def _():
            tokacc[...] += contrib

        @pl.when(e == NL - 1)
        def _():
            out_ref[...] = tokacc[...]


def _experts(x, topk_ids, topk_w, eg, eu, ed):
    grid = (T // TB, NL, IT)
    return pl.pallas_call(
        _expert_body,
        grid=grid,
        in_specs=[
            pl.BlockSpec((TB, K), lambda nt, e, it: (nt, 0)),
            pl.BlockSpec((TB, K), lambda nt, e, it: (nt, 0)),
            pl.BlockSpec((TB, D), lambda nt, e, it: (nt, 0)),
            pl.BlockSpec((1, TI, D), lambda nt, e, it: (e, it, 0)),
            pl.BlockSpec((1, TI, D), lambda nt, e, it: (e, it, 0)),
            pl.BlockSpec((1, D, TI), lambda nt, e, it: (e, 0, it)),
        ],
        out_specs=pl.BlockSpec((TB, D), lambda nt, e, it: (nt, 0)),
        out_shape=jax.ShapeDtypeStruct((T, D), jnp.float32),
        scratch_shapes=[
            pltpu.VMEM((TB, D), jnp.float32),
            pltpu.VMEM((TB, D), jnp.float32),
        ],
    )(topk_ids, topk_w, x, eg, eu, ed)


def kernel(x, gate_w, expert_gate, expert_up, expert_down):
    x_flat = x.reshape(-1, x.shape[-1])
    topk_ids, topk_w, aux = _router(x_flat, gate_w)
    out = _experts(x_flat, topk_ids, topk_w, expert_gate, expert_up,
                   expert_down)
    return out.reshape(x.shape), aux[0, 0]
